# baseline (speedup 1.0000x reference)
import functools

import jax
import jax.numpy as jnp
from jax import lax
from jax.experimental import pallas as pl
from jax.experimental.pallas import tpu as pltpu
from jax.experimental.pallas import tpu_sc as plsc

N_FIELDS = 26
VOCAB = 100000
EMB = 32
N_CONT = 13
BATCH = 16384
OUT_W = N_FIELDS * EMB + N_CONT

NC = 2
NS = 16
NW = NC * NS
BPW = BATCH // NW
GATHER_CHUNK = 128
NCHUNK = BPW // GATHER_CHUNK


def _body(xcatT_hbm, xcont_hbm, table_hbm, mean_hbm, std_hbm, out_hbm,
          idx_v, xcont_v, cont_v, rows_a, rows_b, mean_v, std_v,
          sem_in, sem_g, sem_w):
    wid = lax.axis_index("s") * NC + lax.axis_index("c")
    base = wid * BPW

    cps = []
    for g in range(4):
        cps.append(pltpu.async_copy(
            xcatT_hbm.at[pl.ds(8 * g, 8), pl.ds(base, BPW)], idx_v[g], sem_in))
    cps.append(pltpu.async_copy(
        xcont_hbm.at[pl.ds(base, BPW), :], xcont_v, sem_in))
    cps.append(pltpu.async_copy(mean_hbm, mean_v, sem_in))
    cps.append(pltpu.async_copy(std_hbm, std_v, sem_in))
    for cp in cps:
        cp.wait()

    rows = (rows_a, rows_b)
    prev_writes = [None, None]
    HALF = BPW // 2
    for u in range(2 * N_FIELDS):
        f, h = u // 2, u % 2
        g, r = f // 8, f % 8
        buf = rows[u % 2]
        if prev_writes[u % 2] is not None:
            prev_writes[u % 2].wait()
        gathers = []
        for k in range(HALF // GATHER_CHUNK):
            co = h * HALF + k * GATHER_CHUNK
            gathers.append(pltpu.async_copy(
                table_hbm.at[f].at[idx_v[g].at[r, pl.ds(co, GATHER_CHUNK)]],
                buf.at[pl.ds(k * GATHER_CHUNK, GATHER_CHUNK), :],
                sem_g))
        for cp in gathers:
            cp.wait()
        prev_writes[u % 2] = pltpu.async_copy(
            buf.at[:, pl.ds(0, EMB)],
            out_hbm.at[pl.ds(base + h * HALF, HALF), pl.ds(f * EMB, EMB)],
            sem_w)

    lane = lax.iota(jnp.int32, 16)
    for c in range(N_CONT):
        colc = jnp.full((16,), c, dtype=jnp.int32)
        m = mean_v[c, :]
        s = std_v[c, :]
        def cont_body(i, _, colc=colc, m=m, s=s):
            row = lane + i * 16
            v = plsc.load_gather(xcont_v, [row, colc])
            plsc.store_scatter(cont_v, [row, colc], (v - m) / s)
            return _
        lax.fori_loop(0, BPW // 16, cont_body, 0)
    cp_c = pltpu.async_copy(
        cont_v, out_hbm.at[pl.ds(base, BPW), pl.ds(N_FIELDS * EMB, N_CONT)],
        sem_w)
    for cp in prev_writes:
        if cp is not None:
            cp.wait()
    cp_c.wait()


@jax.jit
def _sc_call(xcatT, xcont, table_pad, mean_b, std_b):
    kfn = pl.kernel(
        _body,
        out_type=jax.ShapeDtypeStruct((BATCH, OUT_W), jnp.float32),
        mesh=plsc.VectorSubcoreMesh(core_axis_name="c", subcore_axis_name="s"),
        scratch_types=[
            [pltpu.VMEM((8, BPW), jnp.int32) for _ in range(4)],
            pltpu.VMEM((BPW, N_CONT), jnp.float32),
            pltpu.VMEM((BPW, N_CONT), jnp.float32),
            pltpu.VMEM((BPW // 2, 128), jnp.float32),
            pltpu.VMEM((BPW // 2, 128), jnp.float32),
            pltpu.VMEM((16, 16), jnp.float32),
            pltpu.VMEM((16, 16), jnp.float32),
            pltpu.SemaphoreType.DMA,
            pltpu.SemaphoreType.DMA,
            pltpu.SemaphoreType.DMA,
        ],
        compiler_params=pltpu.CompilerParams(
            use_tc_tiling_on_sc=False, needs_layout_passes=False),
    )
    return kfn(xcatT, xcont, table_pad, mean_b, std_b)


def kernel(x_cat, x_cont, tables, cont_mean, cont_std):
    table_pad = jnp.pad(tables, ((0, 0), (0, 0), (0, 128 - EMB)))
    xcatT = jnp.pad(x_cat.T, ((0, 32 - N_FIELDS), (0, 0)))
    mean_b = jnp.broadcast_to(
        jnp.pad(cont_mean, (0, 16 - N_CONT))[:, None], (16, 16))
    std_b = jnp.broadcast_to(
        jnp.pad(cont_std, (0, 16 - N_CONT), constant_values=1.0)[:, None],
        (16, 16))
    return _sc_call(xcatT, x_cont, table_pad, mean_b, std_b)


# speedup vs baseline: 1.0705x; 1.0705x over previous
import functools

import jax
import jax.numpy as jnp
from jax import lax
from jax.experimental import pallas as pl
from jax.experimental.pallas import tpu as pltpu
from jax.experimental.pallas import tpu_sc as plsc

N_FIELDS = 26
VOCAB = 100000
EMB = 32
N_CONT = 13
BATCH = 16384
OUT_W = N_FIELDS * EMB + N_CONT

NC = 2
NS = 16
NW = NC * NS
BPW = BATCH // NW
GATHER_CHUNK = 128
NCHUNK = BPW // GATHER_CHUNK


def _body(xcatT_hbm, xcontT_hbm, table_hbm, mean_hbm, std_hbm, out_hbm,
          idx_v, xcont_v, cont_v, rows_a, rows_b, mean_v, std_v,
          sem_in, sem_g, sem_w):
    wid = lax.axis_index("s") * NC + lax.axis_index("c")
    base = wid * BPW

    cps = []
    for g in range(4):
        cps.append(pltpu.async_copy(
            xcatT_hbm.at[pl.ds(8 * g, 8), pl.ds(base, BPW)], idx_v[g], sem_in))
    cps.append(pltpu.async_copy(
        xcontT_hbm.at[:, pl.ds(base, BPW)], xcont_v, sem_in))
    cps.append(pltpu.async_copy(mean_hbm, mean_v, sem_in))
    cps.append(pltpu.async_copy(std_hbm, std_v, sem_in))
    for cp in cps:
        cp.wait()

    rows = (rows_a, rows_b)
    prev_write = [None, None]
    for f in range(N_FIELDS):
        g, r = f // 8, f % 8
        buf = rows[f % 2]
        if prev_write[f % 2] is not None:
            prev_write[f % 2].wait()
        gathers = []
        for k in range(NCHUNK):
            gathers.append(pltpu.async_copy(
                table_hbm.at[f].at[idx_v[g].at[r, pl.ds(k * GATHER_CHUNK,
                                                        GATHER_CHUNK)]],
                buf.at[pl.ds(k * GATHER_CHUNK, GATHER_CHUNK), :],
                sem_g))
        for cp in gathers:
            cp.wait()
        prev_write[f % 2] = pltpu.async_copy(
            buf, out_hbm.at[pl.ds(base, BPW), pl.ds(f * EMB, EMB)], sem_w)

    lane = lax.iota(jnp.int32, 16)
    for c in range(N_CONT):
        colc = jnp.full((16,), c, dtype=jnp.int32)
        m = mean_v[c, :]
        s = std_v[c, :]
        def cont_body(i, _, c=c, colc=colc, m=m, s=s):
            row = lane + i * 16
            v = xcont_v[c, pl.ds(i * 16, 16)]
            plsc.store_scatter(cont_v, [row, colc], (v - m) / s)
            return _
        lax.fori_loop(0, BPW // 16, cont_body, 0)
    cp_c = pltpu.async_copy(
        cont_v, out_hbm.at[pl.ds(base, BPW), pl.ds(N_FIELDS * EMB, N_CONT)],
        sem_w)
    for cp in prev_write:
        if cp is not None:
            cp.wait()
    cp_c.wait()


@jax.jit
def _sc_call(xcatT, xcontT, table3d, mean_b, std_b):
    kfn = pl.kernel(
        _body,
        out_type=jax.ShapeDtypeStruct((BATCH, OUT_W), jnp.float32),
        mesh=plsc.VectorSubcoreMesh(core_axis_name="c", subcore_axis_name="s"),
        scratch_types=[
            [pltpu.VMEM((8, BPW), jnp.int32) for _ in range(4)],
            pltpu.VMEM((16, BPW), jnp.float32),
            pltpu.VMEM((BPW, N_CONT), jnp.float32),
            pltpu.VMEM((BPW, EMB), jnp.float32),
            pltpu.VMEM((BPW, EMB), jnp.float32),
            pltpu.VMEM((N_CONT, 16), jnp.float32),
            pltpu.VMEM((N_CONT, 16), jnp.float32),
            pltpu.SemaphoreType.DMA,
            pltpu.SemaphoreType.DMA,
            pltpu.SemaphoreType.DMA,
        ],
        compiler_params=pltpu.CompilerParams(
            use_tc_tiling_on_sc=False, needs_layout_passes=False),
    )
    return kfn(xcatT, xcontT, table3d, mean_b, std_b)


def kernel(x_cat, x_cont, tables, cont_mean, cont_std):
    xcatT = jnp.pad(x_cat.T, ((0, 32 - N_FIELDS), (0, 0)))
    xcontT = jnp.pad(x_cont.T, ((0, 16 - N_CONT), (0, 0)))
    mean_b = jnp.broadcast_to(cont_mean[:, None], (N_CONT, 16))
    std_b = jnp.broadcast_to(cont_std[:, None], (N_CONT, 16))
    return _sc_call(xcatT, xcontT, tables, mean_b, std_b)
